# baseline (speedup 1.0000x reference)
import jax
import jax.numpy as jnp
from jax.experimental import pallas as pl
from jax.experimental.pallas import tpu as pltpu

_TM = 512


def _energy_body(idx_ref, pr_ref, pc_ref, se_ref, adj_ref, pair_ref, atom_ref):
    i = pl.program_id(1)

    @pl.when(i == 0)
    def _init():
        pair_ref[...] = jnp.zeros_like(pair_ref)
        atom_ref[...] = jnp.zeros_like(atom_ref)

    d2 = jax.lax.dot_general(pr_ref[0], pc_ref[0], (((1,), (0,)), ((), ())),
                             preferred_element_type=jnp.float32)
    dist = jnp.sqrt(jnp.maximum(d2, 0.0))
    t = adj_ref[0].astype(jnp.float32) * dist
    pair_ref[...] = pair_ref[...] + jnp.sum(t)

    onehot = (jax.lax.broadcasted_iota(jnp.int32, (idx_ref.shape[1], 128), 1)
              == idx_ref[0]).astype(jnp.float32)
    ae = jnp.dot(onehot, se_ref[...], preferred_element_type=jnp.float32)
    atom_ref[...] = atom_ref[...] + jnp.sum(ae)


def kernel(node_indices, positions, adjacency, mask, species_energy,
           pair_weight):
    B, N = node_indices.shape
    S = species_energy.shape[0]
    TM = _TM

    maskf = mask.astype(jnp.float32)
    r2 = jnp.sum(positions * positions, axis=-1)
    ones = jnp.ones((B, N), jnp.float32)
    zero = jnp.zeros((B, N), jnp.float32)
    pr_aug = maskf[:, :, None] * jnp.stack(
        [positions[..., 0], positions[..., 1], positions[..., 2],
         r2, ones, zero, zero, zero], axis=-1)
    pc_aug = maskf[:, None, :] * jnp.stack(
        [-2.0 * positions[..., 0], -2.0 * positions[..., 1],
         -2.0 * positions[..., 2], ones, r2, zero, zero, zero], axis=1)

    idx2 = jnp.where(mask, node_indices, 127).astype(jnp.int32)
    idx2 = idx2.reshape(B, N, 1)
    se = jnp.zeros((128, 1), jnp.float32).at[:S, 0].set(species_energy)

    grid = (B, N // TM)
    pair, atom = pl.pallas_call(
        _energy_body,
        grid=grid,
        in_specs=[
            pl.BlockSpec((1, TM, 1), lambda b, i: (b, i, 0)),
            pl.BlockSpec((1, TM, 8), lambda b, i: (b, i, 0)),
            pl.BlockSpec((1, 8, N), lambda b, i: (b, 0, 0)),
            pl.BlockSpec((128, 1), lambda b, i: (0, 0)),
            pl.BlockSpec((1, TM, N), lambda b, i: (b, i, 0)),
        ],
        out_specs=[
            pl.BlockSpec((1, 8, 128), lambda b, i: (b, 0, 0)),
            pl.BlockSpec((1, 8, 128), lambda b, i: (b, 0, 0)),
        ],
        out_shape=[
            jax.ShapeDtypeStruct((B, 8, 128), jnp.float32),
            jax.ShapeDtypeStruct((B, 8, 128), jnp.float32),
        ],
        compiler_params=pltpu.CompilerParams(
            dimension_semantics=("parallel", "arbitrary")),
    )(idx2, pr_aug, pc_aug, se, adjacency)

    return atom[:, 0, 0] + pair_weight * pair[:, 0, 0]


# speedup vs baseline: 1.5650x; 1.5650x over previous
import jax
import jax.numpy as jnp
from jax.experimental import pallas as pl
from jax.experimental.pallas import tpu as pltpu

_TM = 512


def _energy_body(idx_ref, pr_ref, pc_ref, mr_ref, mc_ref, se_ref, adj_ref,
                 pair_ref, atom_ref):
    i = pl.program_id(1)

    @pl.when(i == 0)
    def _init():
        pair_ref[...] = jnp.zeros_like(pair_ref)
        atom_ref[...] = jnp.zeros_like(atom_ref)

    pr = pr_ref[0]
    pc = pc_ref[0]
    mr = mr_ref[0]
    mc = mc_ref[0]

    r2r = jnp.sum(pr * pr, axis=1, keepdims=True)
    r2c = jnp.sum(pc * pc, axis=0, keepdims=True)
    pr_aug = jnp.concatenate([pr, r2r, jnp.ones_like(r2r)], axis=1) * mr
    pc_aug = jnp.concatenate([-2.0 * pc, jnp.ones_like(r2c), r2c],
                             axis=0) * mc
    d2 = jax.lax.dot_general(pr_aug, pc_aug, (((1,), (0,)), ((), ())),
                             preferred_element_type=jnp.float32)
    d2 = jnp.maximum(d2, 0.0)
    dist = d2 * jax.lax.rsqrt(jnp.maximum(d2, 1e-35))
    t = adj_ref[0].astype(jnp.float32) * dist
    pair_ref[...] = pair_ref[...] + jnp.sum(t)

    onehot = (jax.lax.broadcasted_iota(jnp.int32, (idx_ref.shape[1], 128), 1)
              == idx_ref[0]).astype(jnp.float32)
    ae = jnp.dot(onehot, se_ref[...], preferred_element_type=jnp.float32)
    atom_ref[...] = atom_ref[...] + jnp.sum(ae)


def kernel(node_indices, positions, adjacency, mask, species_energy,
           pair_weight):
    B, N = node_indices.shape
    S = species_energy.shape[0]
    TM = _TM

    maskf = mask.astype(jnp.float32)
    mask_row = maskf.reshape(B, N, 1)
    mask_col = maskf.reshape(B, 1, N)
    pos_c = positions.transpose(0, 2, 1)
    idx2 = jnp.where(mask, node_indices, 127).astype(jnp.int32)
    idx2 = idx2.reshape(B, N, 1)
    se = jnp.zeros((128, 1), jnp.float32).at[:S, 0].set(species_energy)

    grid = (B, N // TM)
    pair, atom = pl.pallas_call(
        _energy_body,
        grid=grid,
        in_specs=[
            pl.BlockSpec((1, TM, 1), lambda b, i: (b, i, 0)),
            pl.BlockSpec((1, TM, 3), lambda b, i: (b, i, 0)),
            pl.BlockSpec((1, 3, N), lambda b, i: (b, 0, 0)),
            pl.BlockSpec((1, TM, 1), lambda b, i: (b, i, 0)),
            pl.BlockSpec((1, 1, N), lambda b, i: (b, 0, 0)),
            pl.BlockSpec((128, 1), lambda b, i: (0, 0)),
            pl.BlockSpec((1, TM, N), lambda b, i: (b, i, 0)),
        ],
        out_specs=[
            pl.BlockSpec((1, 8, 128), lambda b, i: (b, 0, 0)),
            pl.BlockSpec((1, 8, 128), lambda b, i: (b, 0, 0)),
        ],
        out_shape=[
            jax.ShapeDtypeStruct((B, 8, 128), jnp.float32),
            jax.ShapeDtypeStruct((B, 8, 128), jnp.float32),
        ],
        compiler_params=pltpu.CompilerParams(
            dimension_semantics=("parallel", "arbitrary")),
    )(idx2, positions, pos_c, mask_row, mask_col, se, adjacency)

    return atom[:, 0, 0] + pair_weight * pair[:, 0, 0]


# speedup vs baseline: 1.7217x; 1.1001x over previous
import jax
import jax.numpy as jnp
from jax.experimental import pallas as pl
from jax.experimental.pallas import tpu as pltpu

_TM = 1024


def _energy_body(idx_ref, pr_ref, pc_ref, mr_ref, mc_ref, se_ref, adj_ref,
                 pair_ref, atom_ref):
    i = pl.program_id(1)

    @pl.when(i == 0)
    def _init():
        pair_ref[...] = jnp.zeros_like(pair_ref)
        atom_ref[...] = jnp.zeros_like(atom_ref)

    pr = pr_ref[0]
    pc = pc_ref[0]
    mr = mr_ref[0]
    mc = mc_ref[0]

    r2r = jnp.sum(pr * pr, axis=1, keepdims=True)
    r2c = jnp.sum(pc * pc, axis=0, keepdims=True)
    pr_aug = jnp.concatenate([pr, r2r, jnp.ones_like(r2r)], axis=1) * mr
    pc_aug = jnp.concatenate([-2.0 * pc, jnp.ones_like(r2c), r2c],
                             axis=0) * mc
    d2 = jax.lax.dot_general(pr_aug, pc_aug, (((1,), (0,)), ((), ())),
                             preferred_element_type=jnp.float32)
    d2 = jnp.maximum(d2, 0.0)
    dist = d2 * jax.lax.rsqrt(jnp.maximum(d2, 1e-35))
    t = adj_ref[0].astype(jnp.float32) * dist
    pair_ref[...] = pair_ref[...] + jnp.sum(t)

    onehot = (jax.lax.broadcasted_iota(jnp.int32, (idx_ref.shape[1], 128), 1)
              == idx_ref[0]).astype(jnp.float32)
    ae = jnp.dot(onehot, se_ref[...], preferred_element_type=jnp.float32)
    atom_ref[...] = atom_ref[...] + jnp.sum(ae)


def kernel(node_indices, positions, adjacency, mask, species_energy,
           pair_weight):
    B, N = node_indices.shape
    S = species_energy.shape[0]
    TM = _TM

    maskf = mask.astype(jnp.float32)
    mask_row = maskf.reshape(B, N, 1)
    mask_col = maskf.reshape(B, 1, N)
    pos_c = positions.transpose(0, 2, 1)
    idx2 = jnp.where(mask, node_indices, 127).astype(jnp.int32)
    idx2 = idx2.reshape(B, N, 1)
    se = jnp.zeros((128, 1), jnp.float32).at[:S, 0].set(species_energy)

    grid = (B, N // TM)
    pair, atom = pl.pallas_call(
        _energy_body,
        grid=grid,
        in_specs=[
            pl.BlockSpec((1, TM, 1), lambda b, i: (b, i, 0)),
            pl.BlockSpec((1, TM, 3), lambda b, i: (b, i, 0)),
            pl.BlockSpec((1, 3, N), lambda b, i: (b, 0, 0)),
            pl.BlockSpec((1, TM, 1), lambda b, i: (b, i, 0)),
            pl.BlockSpec((1, 1, N), lambda b, i: (b, 0, 0)),
            pl.BlockSpec((128, 1), lambda b, i: (0, 0)),
            pl.BlockSpec((1, TM, N), lambda b, i: (b, i, 0)),
        ],
        out_specs=[
            pl.BlockSpec((1, 8, 128), lambda b, i: (b, 0, 0)),
            pl.BlockSpec((1, 8, 128), lambda b, i: (b, 0, 0)),
        ],
        out_shape=[
            jax.ShapeDtypeStruct((B, 8, 128), jnp.float32),
            jax.ShapeDtypeStruct((B, 8, 128), jnp.float32),
        ],
        compiler_params=pltpu.CompilerParams(
            dimension_semantics=("parallel", "arbitrary")),
    )(idx2, positions, pos_c, mask_row, mask_col, se, adjacency)

    return atom[:, 0, 0] + pair_weight * pair[:, 0, 0]


# speedup vs baseline: 1.7726x; 1.0296x over previous
import jax
import jax.numpy as jnp
from jax.experimental import pallas as pl
from jax.experimental.pallas import tpu as pltpu

_TM = 2048


def _energy_body(idx_ref, pr_ref, pc_ref, mr_ref, mc_ref, se_ref, adj_ref,
                 pair_ref, atom_ref):
    i = pl.program_id(1)

    @pl.when(i == 0)
    def _init():
        pair_ref[...] = jnp.zeros_like(pair_ref)
        atom_ref[...] = jnp.zeros_like(atom_ref)

    pr = pr_ref[0]
    pc = pc_ref[0]
    mr = mr_ref[0]
    mc = mc_ref[0]

    r2r = jnp.sum(pr * pr, axis=1, keepdims=True)
    r2c = jnp.sum(pc * pc, axis=0, keepdims=True)
    pr_aug = jnp.concatenate([pr, r2r, jnp.ones_like(r2r)], axis=1) * mr
    pc_aug = jnp.concatenate([-2.0 * pc, jnp.ones_like(r2c), r2c],
                             axis=0) * mc
    d2 = jax.lax.dot_general(pr_aug, pc_aug, (((1,), (0,)), ((), ())),
                             preferred_element_type=jnp.float32)
    d2 = jnp.maximum(d2, 0.0)
    dist = d2 * jax.lax.rsqrt(jnp.maximum(d2, 1e-35))
    t = adj_ref[0].astype(jnp.float32) * dist
    pair_ref[...] = pair_ref[...] + jnp.sum(t)

    onehot = (jax.lax.broadcasted_iota(jnp.int32, (idx_ref.shape[1], 128), 1)
              == idx_ref[0]).astype(jnp.float32)
    ae = jnp.dot(onehot, se_ref[...], preferred_element_type=jnp.float32)
    atom_ref[...] = atom_ref[...] + jnp.sum(ae)


def kernel(node_indices, positions, adjacency, mask, species_energy,
           pair_weight):
    B, N = node_indices.shape
    S = species_energy.shape[0]
    TM = _TM

    maskf = mask.astype(jnp.float32)
    mask_row = maskf.reshape(B, N, 1)
    mask_col = maskf.reshape(B, 1, N)
    pos_c = positions.transpose(0, 2, 1)
    idx2 = jnp.where(mask, node_indices, 127).astype(jnp.int32)
    idx2 = idx2.reshape(B, N, 1)
    se = jnp.zeros((128, 1), jnp.float32).at[:S, 0].set(species_energy)

    grid = (B, N // TM)
    pair, atom = pl.pallas_call(
        _energy_body,
        grid=grid,
        in_specs=[
            pl.BlockSpec((1, TM, 1), lambda b, i: (b, i, 0)),
            pl.BlockSpec((1, TM, 3), lambda b, i: (b, i, 0)),
            pl.BlockSpec((1, 3, N), lambda b, i: (b, 0, 0)),
            pl.BlockSpec((1, TM, 1), lambda b, i: (b, i, 0)),
            pl.BlockSpec((1, 1, N), lambda b, i: (b, 0, 0)),
            pl.BlockSpec((128, 1), lambda b, i: (0, 0)),
            pl.BlockSpec((1, TM, N), lambda b, i: (b, i, 0)),
        ],
        out_specs=[
            pl.BlockSpec((1, 8, 128), lambda b, i: (b, 0, 0)),
            pl.BlockSpec((1, 8, 128), lambda b, i: (b, 0, 0)),
        ],
        out_shape=[
            jax.ShapeDtypeStruct((B, 8, 128), jnp.float32),
            jax.ShapeDtypeStruct((B, 8, 128), jnp.float32),
        ],
        compiler_params=pltpu.CompilerParams(
            dimension_semantics=("parallel", "arbitrary")),
    )(idx2, positions, pos_c, mask_row, mask_col, se, adjacency)

    return atom[:, 0, 0] + pair_weight * pair[:, 0, 0]


# speedup vs baseline: 1.8636x; 1.0514x over previous
import jax
import jax.numpy as jnp
from jax.experimental import pallas as pl
from jax.experimental.pallas import tpu as pltpu

_TM = 2048


def _energy_body(idx_ref, pr_ref, pc_ref, mr_ref, mc_ref, se_ref, adj_ref,
                 adj2_ref, pair_ref, atom_ref):
    i = pl.program_id(1)

    @pl.when(i == 0)
    def _init():
        pair_ref[...] = jnp.zeros_like(pair_ref)
        atom_ref[...] = jnp.zeros_like(atom_ref)

    pr = pr_ref[0]
    pc = pc_ref[0]
    mr = mr_ref[0]
    mc = mc_ref[0]

    r2r = jnp.sum(pr * pr, axis=1, keepdims=True)
    r2c = jnp.sum(pc * pc, axis=0, keepdims=True)
    pr_aug = jnp.concatenate([pr, r2r, jnp.ones_like(r2r)], axis=1) * mr
    pc_aug = jnp.concatenate([-2.0 * pc, jnp.ones_like(r2c), r2c],
                             axis=0) * mc
    d2 = jax.lax.dot_general(pr_aug, pc_aug, (((1,), (0,)), ((), ())),
                             preferred_element_type=jnp.float32)
    d2 = jnp.maximum(d2, 0.0)
    dist = d2 * jax.lax.rsqrt(jnp.maximum(d2, 1e-35))
    half = dist.shape[1] // 2
    t = (adj_ref[0].astype(jnp.float32) * dist[:, :half]
         + adj2_ref[0].astype(jnp.float32) * dist[:, half:])
    pair_ref[...] = pair_ref[...] + jnp.sum(t)

    onehot = (jax.lax.broadcasted_iota(jnp.int32, (idx_ref.shape[1], 128), 1)
              == idx_ref[0]).astype(jnp.float32)
    ae = jnp.dot(onehot, se_ref[...], preferred_element_type=jnp.float32)
    atom_ref[...] = atom_ref[...] + jnp.sum(ae)


def kernel(node_indices, positions, adjacency, mask, species_energy,
           pair_weight):
    B, N = node_indices.shape
    S = species_energy.shape[0]
    TM = _TM

    maskf = mask.astype(jnp.float32)
    mask_row = maskf.reshape(B, N, 1)
    mask_col = maskf.reshape(B, 1, N)
    pos_c = positions.transpose(0, 2, 1)
    idx2 = jnp.where(mask, node_indices, 127).astype(jnp.int32)
    idx2 = idx2.reshape(B, N, 1)
    se = jnp.zeros((128, 1), jnp.float32).at[:S, 0].set(species_energy)

    grid = (B, N // TM)
    pair, atom = pl.pallas_call(
        _energy_body,
        grid=grid,
        in_specs=[
            pl.BlockSpec((1, TM, 1), lambda b, i: (b, i, 0)),
            pl.BlockSpec((1, TM, 3), lambda b, i: (b, i, 0)),
            pl.BlockSpec((1, 3, N), lambda b, i: (b, 0, 0)),
            pl.BlockSpec((1, TM, 1), lambda b, i: (b, i, 0)),
            pl.BlockSpec((1, 1, N), lambda b, i: (b, 0, 0)),
            pl.BlockSpec((128, 1), lambda b, i: (0, 0)),
            pl.BlockSpec((1, TM, N // 2), lambda b, i: (b, i, 0)),
            pl.BlockSpec((1, TM, N // 2), lambda b, i: (b, i, 1)),
        ],
        out_specs=[
            pl.BlockSpec((1, 8, 128), lambda b, i: (b, 0, 0)),
            pl.BlockSpec((1, 8, 128), lambda b, i: (b, 0, 0)),
        ],
        out_shape=[
            jax.ShapeDtypeStruct((B, 8, 128), jnp.float32),
            jax.ShapeDtypeStruct((B, 8, 128), jnp.float32),
        ],
        compiler_params=pltpu.CompilerParams(
            dimension_semantics=("parallel", "arbitrary")),
    )(idx2, positions, pos_c, mask_row, mask_col, se, adjacency, adjacency)

    return atom[:, 0, 0] + pair_weight * pair[:, 0, 0]


# speedup vs baseline: 1.8944x; 1.0166x over previous
import jax
import jax.numpy as jnp
from jax.experimental import pallas as pl
from jax.experimental.pallas import tpu as pltpu

_TM = 2048


def _energy_body(idx_ref, pr_ref, pc_ref, mr_ref, mc_ref, se_ref,
                 adj0_ref, adj1_ref, adj2_ref, adj3_ref,
                 pair_ref, atom_ref):
    i = pl.program_id(1)

    @pl.when(i == 0)
    def _init():
        pair_ref[...] = jnp.zeros_like(pair_ref)
        atom_ref[...] = jnp.zeros_like(atom_ref)

    pr = pr_ref[0]
    pc = pc_ref[0]
    mr = mr_ref[0]
    mc = mc_ref[0]

    r2r = jnp.sum(pr * pr, axis=1, keepdims=True)
    r2c = jnp.sum(pc * pc, axis=0, keepdims=True)
    pr_aug = jnp.concatenate([pr, r2r, jnp.ones_like(r2r)], axis=1) * mr
    pc_aug = jnp.concatenate([-2.0 * pc, jnp.ones_like(r2c), r2c],
                             axis=0) * mc
    d2 = jax.lax.dot_general(pr_aug, pc_aug, (((1,), (0,)), ((), ())),
                             preferred_element_type=jnp.float32)
    d2 = jnp.maximum(d2, 0.0)
    dist = d2 * jax.lax.rsqrt(jnp.maximum(d2, 1e-35))
    q = dist.shape[1] // 4
    t = (adj0_ref[0].astype(jnp.float32) * dist[:, 0 * q:1 * q]
         + adj1_ref[0].astype(jnp.float32) * dist[:, 1 * q:2 * q]
         + adj2_ref[0].astype(jnp.float32) * dist[:, 2 * q:3 * q]
         + adj3_ref[0].astype(jnp.float32) * dist[:, 3 * q:4 * q])
    pair_ref[...] = pair_ref[...] + jnp.sum(t)

    onehot = (jax.lax.broadcasted_iota(jnp.int32, (idx_ref.shape[1], 128), 1)
              == idx_ref[0]).astype(jnp.float32)
    ae = jnp.dot(onehot, se_ref[...], preferred_element_type=jnp.float32)
    atom_ref[...] = atom_ref[...] + jnp.sum(ae)


def kernel(node_indices, positions, adjacency, mask, species_energy,
           pair_weight):
    B, N = node_indices.shape
    S = species_energy.shape[0]
    TM = _TM

    maskf = mask.astype(jnp.float32)
    mask_row = maskf.reshape(B, N, 1)
    mask_col = maskf.reshape(B, 1, N)
    pos_c = positions.transpose(0, 2, 1)
    idx2 = jnp.where(mask, node_indices, 127).astype(jnp.int32)
    idx2 = idx2.reshape(B, N, 1)
    se = jnp.zeros((128, 1), jnp.float32).at[:S, 0].set(species_energy)

    grid = (B, N // TM)
    pair, atom = pl.pallas_call(
        _energy_body,
        grid=grid,
        in_specs=[
            pl.BlockSpec((1, TM, 1), lambda b, i: (b, i, 0)),
            pl.BlockSpec((1, TM, 3), lambda b, i: (b, i, 0)),
            pl.BlockSpec((1, 3, N), lambda b, i: (b, 0, 0)),
            pl.BlockSpec((1, TM, 1), lambda b, i: (b, i, 0)),
            pl.BlockSpec((1, 1, N), lambda b, i: (b, 0, 0)),
            pl.BlockSpec((128, 1), lambda b, i: (0, 0)),
            pl.BlockSpec((1, TM, N // 4), lambda b, i: (b, i, 0)),
            pl.BlockSpec((1, TM, N // 4), lambda b, i: (b, i, 1)),
            pl.BlockSpec((1, TM, N // 4), lambda b, i: (b, i, 2)),
            pl.BlockSpec((1, TM, N // 4), lambda b, i: (b, i, 3)),
        ],
        out_specs=[
            pl.BlockSpec((1, 8, 128), lambda b, i: (b, 0, 0)),
            pl.BlockSpec((1, 8, 128), lambda b, i: (b, 0, 0)),
        ],
        out_shape=[
            jax.ShapeDtypeStruct((B, 8, 128), jnp.float32),
            jax.ShapeDtypeStruct((B, 8, 128), jnp.float32),
        ],
        compiler_params=pltpu.CompilerParams(
            dimension_semantics=("parallel", "arbitrary")),
    )(idx2, positions, pos_c, mask_row, mask_col, se,
      adjacency, adjacency, adjacency, adjacency)

    return atom[:, 0, 0] + pair_weight * pair[:, 0, 0]
